# baseline (speedup 1.0000x reference)
import functools

import jax
import jax.numpy as jnp
from jax import lax
from jax.experimental import pallas as pl
from jax.experimental.pallas import tpu as pltpu
from jax.experimental.pallas import tpu_sc as plsc

F = 26
V = 100000
D = 64
B = 4096
NC, NS = 2, 16
NW = NC * NS
CH = B // NW
NPAIR = F * (F + 1) // 2
PROWS = 8 * NPAIR
PPAD = 2816

BLK = 512


_sc_mesh = plsc.VectorSubcoreMesh(core_axis_name="c", subcore_axis_name="s")


TPF = V // 8
HK = 32


@functools.partial(
    pl.kernel,
    mesh=_sc_mesh,
    out_type=jax.ShapeDtypeStruct((F, B, D), jnp.float32),
    scratch_types=[
        pltpu.VMEM((2, CH), jnp.int32),
        pltpu.VMEM((2, CH, D), jnp.float32),
        pltpu.SemaphoreType.DMA,
        pltpu.SemaphoreType.DMA,
        pltpu.SemaphoreType.DMA,
        pltpu.SemaphoreType.DMA,
    ],
    compiler_params=pltpu.CompilerParams(use_tc_tiling_on_sc=False),
)
def _sc_gather(table_hbm, idx_hbm, out_hbm, idx_v, rows_v, sg0, sg1, so0,
               so1):
    wid = lax.axis_index("s") * NC + lax.axis_index("c")
    base = wid * CH

    def fire(h, f, gsem):
        pltpu.sync_copy(idx_hbm.at[f, pl.ds(base, CH)], idx_v.at[h])
        off = f * V
        for i in range(CH // 16):
            sl = pl.ds(i * 16, 16)
            idx_v[h, sl] = idx_v[h, sl] + off
        pltpu.async_copy(table_hbm.at[idx_v.at[h]], rows_v.at[h], gsem)

    def gwait(h, gsem):
        pltpu.make_async_copy(
            table_hbm.at[idx_v.at[h]], rows_v.at[h], gsem).wait()

    def owait(h, osem):
        pltpu.make_async_copy(
            rows_v.at[h], out_hbm.at[0, pl.ds(0, CH)], osem).wait()

    fire(0, 0, sg0)

    def fbody(f, _):
        f0 = 2 * f
        f1 = f0 + 1

        @pl.when(f > 0)
        def _():
            owait(1, so1)
        fire(1, f1, sg1)
        gwait(0, sg0)
        pltpu.async_copy(rows_v.at[0], out_hbm.at[f0, pl.ds(base, CH)], so0)

        @pl.when(f < F // 2 - 1)
        def _():
            owait(0, so0)
            fire(0, f0 + 2, sg0)
        gwait(1, sg1)
        pltpu.async_copy(rows_v.at[1], out_hbm.at[f1, pl.ds(base, CH)], so1)
        return 0

    lax.fori_loop(0, F // 2, fbody, 0)
    owait(0, so0)
    owait(1, so1)


def _tc_body(dxT, ly, w0, b0, w1, b1, w2, b2, w0a, w0zexp, tb0, tw1, tb1,
             tw2, tb2, out, pstack):
    bf = jnp.bfloat16
    f32 = jnp.float32

    def dot(a, b):
        return jnp.dot(a, b, preferred_element_type=f32)

    h = jnp.maximum(dot(w0[...], dxT[...].astype(bf)) + b0[...], 0.0)
    h = jnp.maximum(dot(w1[...], h.astype(bf)) + b1[...], 0.0)
    xT = jnp.maximum(dot(w2[...], h.astype(bf)) + b2[...], 0.0)

    feats = [xT] + [jnp.transpose(ly[i]) for i in range(F)]
    k = 0
    for n in range(1, F + 1):
        fn = feats[n]
        for m in range(n):
            prod = fn * feats[m]
            s = prod[0:8] + prod[8:16] + prod[16:24] + prod[24:32] \
                + prod[32:40] + prod[40:48] + prod[48:56] + prod[56:64]
            pstack[pl.ds(8 * k, 8), :] = s.astype(bf)
            k += 1
    pstack[pl.ds(PROWS, PPAD - PROWS), :] = jnp.zeros(
        (PPAD - PROWS, BLK), bf)

    y = dot(w0a[...], xT.astype(bf)) + dot(w0zexp[...], pstack[...]) + tb0[...]
    y = jnp.maximum(y, 0.0)
    y = jnp.maximum(dot(tw1[...], y.astype(bf)) + tb1[...], 0.0)
    y = dot(tw2[...], y.astype(bf)) + tb2[...]
    out[...] = jax.nn.sigmoid(y)


def _full(shape):
    return pl.BlockSpec(shape, lambda i: tuple(0 for _ in shape))


def _tc_forward(dxT, ly, *args):
    w_specs = [_full(a.shape) for a in args]
    return pl.pallas_call(
        _tc_body,
        grid=(B // BLK,),
        in_specs=[
            pl.BlockSpec((16, BLK), lambda i: (0, i)),
            pl.BlockSpec((F, BLK, D), lambda i: (0, i, 0)),
            *w_specs,
        ],
        out_specs=pl.BlockSpec((1, BLK), lambda i: (0, i)),
        out_shape=jax.ShapeDtypeStruct((1, B), jnp.float32),
        scratch_shapes=[pltpu.VMEM((PPAD, BLK), jnp.bfloat16)],
    )(dxT, ly, *args)


def kernel(dense_x, lS_i, emb_w, bot_w0, bot_b0, bot_w1, bot_b1, bot_w2,
           bot_b2, top_w0, top_b0, top_w1, top_b1, top_w2, top_b2):
    bf = jnp.bfloat16
    ly = _sc_gather(emb_w.reshape(F * V, D), lS_i)

    dxT = jnp.pad(dense_x, ((0, 0), (0, 3))).T
    w0zexp = jnp.pad(jnp.repeat(top_w0[:, D:], 8, axis=1),
                     ((0, 0), (0, PPAD - PROWS))).astype(bf)
    yt = _tc_forward(
        dxT, ly,
        jnp.pad(bot_w0, ((0, 0), (0, 3))).astype(bf), bot_b0[:, None],
        bot_w1.astype(bf), bot_b1[:, None],
        bot_w2.astype(bf), bot_b2[:, None],
        top_w0[:, :D].astype(bf), w0zexp, top_b0[:, None],
        top_w1.astype(bf), top_b1[:, None],
        top_w2.astype(bf), top_b2[:, None],
    )
    return yt.reshape(B, 1)


# speedup vs baseline: 2.1894x; 2.1894x over previous
import functools

import jax
import jax.numpy as jnp
from jax import lax
from jax.experimental import pallas as pl
from jax.experimental.pallas import tpu as pltpu
from jax.experimental.pallas import tpu_sc as plsc

F = 26
V = 100000
D = 64
B = 4096
NC, NS = 2, 16
NW = NC * NS
CH = B // NW
NPAIR = F * (F + 1) // 2
PROWS = 8 * NPAIR
PPAD = 2816

BLK = 512


_sc_mesh = plsc.VectorSubcoreMesh(core_axis_name="c", subcore_axis_name="s")


TPF = V // 8
HK = 32


@functools.partial(
    pl.kernel,
    mesh=_sc_mesh,
    out_type=jax.ShapeDtypeStruct((F, B, D), jnp.float32),
    scratch_types=[
        pltpu.VMEM((F, CH), jnp.int32),
        pltpu.VMEM((2, HK, 8, D), jnp.float32),
        pltpu.VMEM((CH, D), jnp.float32),
        pltpu.SemaphoreType.DMA,
        pltpu.SemaphoreType.DMA,
        pltpu.SemaphoreType.DMA,
    ],
)
def _sc_gather(table_hbm, idx_hbm, out_hbm, idx_v, tiles_v, rows_v,
               sa, sb, so):
    wid = lax.axis_index("s") * NC + lax.axis_index("c")
    base = wid * CH
    pltpu.sync_copy(idx_hbm.at[:, pl.ds(base, CH)], idx_v)

    def fire(h, f, i0):
        sem = sa if h == 0 else sb
        for g in range(HK // 16):
            t16 = lax.shift_right_logical(
                idx_v[f, pl.ds(i0 + g * 16, 16)], 3) + f * TPF
            for u in range(16):
                pltpu.async_copy(
                    table_hbm.at[t16[u]], tiles_v.at[h, g * 16 + u], sem)

    def drain(h):
        sem = sa if h == 0 else sb
        pltpu.make_async_copy(
            table_hbm.at[pl.ds(0, HK)], tiles_v.at[h], sem).wait()

    def extract(h, f, i0):
        for g in range(HK // 16):
            r16 = jnp.bitwise_and(idx_v[f, pl.ds(i0 + g * 16, 16)], 7)
            for u in range(16):
                k = g * 16 + u
                r = r16[u]
                for j in range(D // 16):
                    sl = pl.ds(j * 16, 16)
                    rows_v[i0 + k, sl] = tiles_v[h, k, r, sl]

    fire(0, 0, 0)

    def fbody(f, _):
        fire(1, f, HK)
        drain(0)

        @pl.when(f > 0)
        def _():
            pltpu.make_async_copy(
                rows_v, out_hbm.at[0, pl.ds(0, CH)], so).wait()
        extract(0, f, 0)
        fire(0, f, 2 * HK)
        drain(1)
        extract(1, f, HK)
        fire(1, f, 3 * HK)
        drain(0)
        extract(0, f, 2 * HK)

        @pl.when(f < F - 1)
        def _():
            fire(0, f + 1, 0)
        drain(1)
        extract(1, f, 3 * HK)
        pltpu.async_copy(rows_v, out_hbm.at[f, pl.ds(base, CH)], so)
        return 0

    lax.fori_loop(0, F, fbody, 0)
    pltpu.make_async_copy(rows_v, out_hbm.at[0, pl.ds(0, CH)], so).wait()


def _tc_body(dxT, ly, w0, b0, w1, b1, w2, b2, w0a, w0zexp, tb0, tw1, tb1,
             tw2, tb2, out, pstack):
    bf = jnp.bfloat16
    f32 = jnp.float32

    def dot(a, b):
        return jnp.dot(a, b, preferred_element_type=f32)

    h = jnp.maximum(dot(w0[...], dxT[...].astype(bf)) + b0[...], 0.0)
    h = jnp.maximum(dot(w1[...], h.astype(bf)) + b1[...], 0.0)
    xT = jnp.maximum(dot(w2[...], h.astype(bf)) + b2[...], 0.0)

    feats = [xT] + [jnp.transpose(ly[i]) for i in range(F)]
    k = 0
    for n in range(1, F + 1):
        fn = feats[n]
        for m in range(n):
            prod = fn * feats[m]
            s = prod[0:8] + prod[8:16] + prod[16:24] + prod[24:32] \
                + prod[32:40] + prod[40:48] + prod[48:56] + prod[56:64]
            pstack[pl.ds(8 * k, 8), :] = s.astype(bf)
            k += 1
    pstack[pl.ds(PROWS, PPAD - PROWS), :] = jnp.zeros(
        (PPAD - PROWS, BLK), bf)

    y = dot(w0a[...], xT.astype(bf)) + dot(w0zexp[...], pstack[...]) + tb0[...]
    y = jnp.maximum(y, 0.0)
    y = jnp.maximum(dot(tw1[...], y.astype(bf)) + tb1[...], 0.0)
    y = dot(tw2[...], y.astype(bf)) + tb2[...]
    out[...] = jax.nn.sigmoid(y)


def _full(shape):
    return pl.BlockSpec(shape, lambda i: tuple(0 for _ in shape))


def _tc_forward(dxT, ly, *args):
    w_specs = [_full(a.shape) for a in args]
    return pl.pallas_call(
        _tc_body,
        grid=(B // BLK,),
        in_specs=[
            pl.BlockSpec((16, BLK), lambda i: (0, i)),
            pl.BlockSpec((F, BLK, D), lambda i: (0, i, 0)),
            *w_specs,
        ],
        out_specs=pl.BlockSpec((1, BLK), lambda i: (0, i)),
        out_shape=jax.ShapeDtypeStruct((1, B), jnp.float32),
        scratch_shapes=[pltpu.VMEM((PPAD, BLK), jnp.bfloat16)],
    )(dxT, ly, *args)


def kernel(dense_x, lS_i, emb_w, bot_w0, bot_b0, bot_w1, bot_b1, bot_w2,
           bot_b2, top_w0, top_b0, top_w1, top_b1, top_w2, top_b2):
    bf = jnp.bfloat16
    ly = _sc_gather(emb_w.reshape(F * TPF, 8, D), lS_i)

    dxT = jnp.pad(dense_x, ((0, 0), (0, 3))).T
    w0zexp = jnp.pad(jnp.repeat(top_w0[:, D:], 8, axis=1),
                     ((0, 0), (0, PPAD - PROWS))).astype(bf)
    yt = _tc_forward(
        dxT, ly,
        jnp.pad(bot_w0, ((0, 0), (0, 3))).astype(bf), bot_b0[:, None],
        bot_w1.astype(bf), bot_b1[:, None],
        bot_w2.astype(bf), bot_b2[:, None],
        top_w0[:, :D].astype(bf), w0zexp, top_b0[:, None],
        top_w1.astype(bf), top_b1[:, None],
        top_w2.astype(bf), top_b2[:, None],
    )
    return yt.reshape(B, 1)


# speedup vs baseline: 2.1916x; 1.0010x over previous
import functools

import jax
import jax.numpy as jnp
from jax import lax
from jax.experimental import pallas as pl
from jax.experimental.pallas import tpu as pltpu
from jax.experimental.pallas import tpu_sc as plsc

F = 26
V = 100000
D = 64
B = 4096
NC, NS = 2, 16
NW = NC * NS
CH = B // NW
NPAIR = F * (F + 1) // 2
PROWS = 8 * NPAIR
PPAD = 2816

BLK = 1024


_sc_mesh = plsc.VectorSubcoreMesh(core_axis_name="c", subcore_axis_name="s")


TPF = V // 8
HK = 32


@functools.partial(
    pl.kernel,
    mesh=_sc_mesh,
    out_type=jax.ShapeDtypeStruct((F, B, D), jnp.float32),
    scratch_types=[
        pltpu.VMEM((F, CH), jnp.int32),
        pltpu.VMEM((2, HK, 8, D), jnp.float32),
        pltpu.VMEM((CH, D), jnp.float32),
        pltpu.SemaphoreType.DMA,
        pltpu.SemaphoreType.DMA,
        pltpu.SemaphoreType.DMA,
    ],
)
def _sc_gather(table_hbm, idx_hbm, out_hbm, idx_v, tiles_v, rows_v,
               sa, sb, so):
    wid = lax.axis_index("s") * NC + lax.axis_index("c")
    base = wid * CH
    pltpu.sync_copy(idx_hbm.at[:, pl.ds(base, CH)], idx_v)

    def fire(h, f, i0):
        sem = sa if h == 0 else sb
        for g in range(HK // 16):
            t16 = lax.shift_right_logical(
                idx_v[f, pl.ds(i0 + g * 16, 16)], 3) + f * TPF
            for u in range(16):
                pltpu.async_copy(
                    table_hbm.at[t16[u]], tiles_v.at[h, g * 16 + u], sem)

    def drain(h):
        sem = sa if h == 0 else sb
        pltpu.make_async_copy(
            table_hbm.at[pl.ds(0, HK)], tiles_v.at[h], sem).wait()

    def extract(h, f, i0):
        for g in range(HK // 16):
            r16 = jnp.bitwise_and(idx_v[f, pl.ds(i0 + g * 16, 16)], 7)
            for u in range(16):
                k = g * 16 + u
                r = r16[u]
                for j in range(D // 16):
                    sl = pl.ds(j * 16, 16)
                    rows_v[i0 + k, sl] = tiles_v[h, k, r, sl]

    fire(0, 0, 0)

    def fbody(f, _):
        fire(1, f, HK)
        drain(0)

        @pl.when(f > 0)
        def _():
            pltpu.make_async_copy(
                rows_v, out_hbm.at[0, pl.ds(0, CH)], so).wait()
        extract(0, f, 0)
        fire(0, f, 2 * HK)
        drain(1)
        extract(1, f, HK)
        fire(1, f, 3 * HK)
        drain(0)
        extract(0, f, 2 * HK)

        @pl.when(f < F - 1)
        def _():
            fire(0, f + 1, 0)
        drain(1)
        extract(1, f, 3 * HK)
        pltpu.async_copy(rows_v, out_hbm.at[f, pl.ds(base, CH)], so)
        return 0

    lax.fori_loop(0, F, fbody, 0)
    pltpu.make_async_copy(rows_v, out_hbm.at[0, pl.ds(0, CH)], so).wait()


def _tc_body(dxT, ly, w0, b0, w1, b1, w2, b2, w0a, w0zexp, tb0, tw1, tb1,
             tw2, tb2, out, pstack):
    bf = jnp.bfloat16
    f32 = jnp.float32

    def dot(a, b):
        return jnp.dot(a, b, preferred_element_type=f32)

    h = jnp.maximum(dot(w0[...], dxT[...].astype(bf)) + b0[...], 0.0)
    h = jnp.maximum(dot(w1[...], h.astype(bf)) + b1[...], 0.0)
    xT = jnp.maximum(dot(w2[...], h.astype(bf)) + b2[...], 0.0)

    feats = [xT] + [jnp.transpose(ly[i]) for i in range(F)]
    k = 0
    for n in range(1, F + 1):
        fn = feats[n]
        for m in range(n):
            prod = fn * feats[m]
            s = prod[0:8] + prod[8:16] + prod[16:24] + prod[24:32] \
                + prod[32:40] + prod[40:48] + prod[48:56] + prod[56:64]
            pstack[pl.ds(8 * k, 8), :] = s.astype(bf)
            k += 1
    pstack[pl.ds(PROWS, PPAD - PROWS), :] = jnp.zeros(
        (PPAD - PROWS, BLK), bf)

    y = dot(w0a[...], xT.astype(bf)) + dot(w0zexp[...], pstack[...]) + tb0[...]
    y = jnp.maximum(y, 0.0)
    y = jnp.maximum(dot(tw1[...], y.astype(bf)) + tb1[...], 0.0)
    y = dot(tw2[...], y.astype(bf)) + tb2[...]
    out[...] = jax.nn.sigmoid(y)


def _full(shape):
    return pl.BlockSpec(shape, lambda i: tuple(0 for _ in shape))


def _tc_forward(dxT, ly, *args):
    w_specs = [_full(a.shape) for a in args]
    return pl.pallas_call(
        _tc_body,
        grid=(B // BLK,),
        in_specs=[
            pl.BlockSpec((16, BLK), lambda i: (0, i)),
            pl.BlockSpec((F, BLK, D), lambda i: (0, i, 0)),
            *w_specs,
        ],
        out_specs=pl.BlockSpec((1, BLK), lambda i: (0, i)),
        out_shape=jax.ShapeDtypeStruct((1, B), jnp.float32),
        scratch_shapes=[pltpu.VMEM((PPAD, BLK), jnp.bfloat16)],
    )(dxT, ly, *args)


def kernel(dense_x, lS_i, emb_w, bot_w0, bot_b0, bot_w1, bot_b1, bot_w2,
           bot_b2, top_w0, top_b0, top_w1, top_b1, top_w2, top_b2):
    bf = jnp.bfloat16
    ly = _sc_gather(emb_w.reshape(F * TPF, 8, D), lS_i)

    dxT = jnp.pad(dense_x, ((0, 0), (0, 3))).T
    w0zexp = jnp.pad(jnp.repeat(top_w0[:, D:], 8, axis=1),
                     ((0, 0), (0, PPAD - PROWS))).astype(bf)
    yt = _tc_forward(
        dxT, ly,
        jnp.pad(bot_w0, ((0, 0), (0, 3))).astype(bf), bot_b0[:, None],
        bot_w1.astype(bf), bot_b1[:, None],
        bot_w2.astype(bf), bot_b2[:, None],
        top_w0[:, :D].astype(bf), w0zexp, top_b0[:, None],
        top_w1.astype(bf), top_b1[:, None],
        top_w2.astype(bf), top_b2[:, None],
    )
    return yt.reshape(B, 1)
